# baseline (speedup 1.0000x reference)
import functools

import jax
import jax.numpy as jnp
from jax import lax
from jax.experimental import pallas as pl
from jax.experimental.pallas import tpu as pltpu
from jax.experimental.pallas import tpu_sc as plsc

NN = 10000
EE = 320000
HH = 128
NC = 2
NS = 16
NW = NC * NS
NPAD = 10240
EPW = 10240
CH = 80
NCHUNK = EPW // CH
RPS = NPAD // NS
BLK = 1000
GRID = NN // BLK

_mesh = plsc.VectorSubcoreMesh(core_axis_name="c", subcore_axis_name="s")


def _deg_body(dsts, zeros1, out, degsh, dstv, onesv):
    c = lax.axis_index("c")
    s = lax.axis_index("s")
    wid = c * NS + s
    for j in range(CH // 16):
        onesv[pl.ds(16 * j, 16)] = jnp.ones((16,), jnp.float32)
    pltpu.sync_copy(zeros1.at[pl.ds(s * RPS, RPS)], degsh.at[pl.ds(s * RPS, RPS)])
    pltpu.sync_copy(dsts.at[wid], dstv)
    plsc.subcore_barrier()

    def chunk(i, carry):
        pltpu.sync_copy(onesv, degsh.at[dstv.at[i]], add=True)
        return carry

    lax.fori_loop(0, NCHUNK, chunk, 0)
    plsc.subcore_barrier()
    pltpu.sync_copy(degsh.at[pl.ds(s * RPS, RPS)], out.at[c, pl.ds(s * RPS, RPS)])


def _deg_call(dst, zeros1):
    kfn = pl.kernel(
        _deg_body,
        out_type=jax.ShapeDtypeStruct((NC, NPAD), jnp.float32),
        mesh=_mesh,
        scratch_types=[
            pltpu.VMEM_SHARED((NPAD,), jnp.float32),
            pltpu.VMEM((NCHUNK, CH), jnp.int32),
            pltpu.VMEM((CH,), jnp.float32),
        ],
    )
    return kfn(dst, zeros1)


def _edge_body(table, packed, zeros2, out, accsh, pk, s0v, d0v, s1v, d1v,
               rows0, rows1, sem0, sem1, ssem0, ssem1):
    c = lax.axis_index("c")
    s = lax.axis_index("s")
    wid = c * NS + s
    pltpu.sync_copy(zeros2.at[pl.ds(s * RPS, RPS)], accsh.at[pl.ds(s * RPS, RPS)])
    pltpu.sync_copy(packed.at[wid], pk)
    plsc.subcore_barrier()

    def unpack(i, sv, dv):
        for j in range(CH // 16):
            p = pk[i, pl.ds(16 * j, 16)]
            sv[pl.ds(16 * j, 16)] = lax.shift_right_logical(p, 14)
            dv[pl.ds(16 * j, 16)] = lax.bitwise_and(p, 16383)

    unpack(0, s0v, d0v)
    pltpu.async_copy(table.at[s0v], rows0, sem0)
    unpack(1, s1v, d1v)
    pltpu.async_copy(table.at[s1v], rows1, sem1)

    def body(k, carry):
        i0 = 2 * k
        pltpu.make_async_copy(table.at[s0v], rows0, sem0).wait()
        pltpu.async_copy(rows0, accsh.at[d0v], ssem0, add=True)
        pltpu.make_async_copy(table.at[s1v], rows1, sem1).wait()
        pltpu.async_copy(rows1, accsh.at[d1v], ssem1, add=True)
        pltpu.make_async_copy(rows0, accsh.at[d0v], ssem0).wait()
        unpack(lax.rem(i0 + 2, NCHUNK), s0v, d0v)
        pltpu.async_copy(table.at[s0v], rows0, sem0)
        pltpu.make_async_copy(rows1, accsh.at[d1v], ssem1).wait()
        unpack(lax.rem(i0 + 3, NCHUNK), s1v, d1v)
        pltpu.async_copy(table.at[s1v], rows1, sem1)
        return carry

    lax.fori_loop(0, NCHUNK // 2, body, 0)
    pltpu.make_async_copy(table.at[s0v], rows0, sem0).wait()
    pltpu.make_async_copy(table.at[s1v], rows1, sem1).wait()
    plsc.subcore_barrier()
    pltpu.sync_copy(accsh.at[pl.ds(s * RPS, RPS)], out.at[c, pl.ds(s * RPS, RPS)])


def _edge_call(table, packed, zeros2):
    kfn = pl.kernel(
        _edge_body,
        out_type=jax.ShapeDtypeStruct((NC, NPAD, HH), jnp.float32),
        mesh=_mesh,
        scratch_types=[
            pltpu.VMEM_SHARED((NPAD, HH), jnp.float32),
            pltpu.VMEM((NCHUNK, CH), jnp.int32),
            pltpu.VMEM((CH,), jnp.int32),
            pltpu.VMEM((CH,), jnp.int32),
            pltpu.VMEM((CH,), jnp.int32),
            pltpu.VMEM((CH,), jnp.int32),
            pltpu.VMEM((CH, HH), jnp.float32),
            pltpu.VMEM((CH, HH), jnp.float32),
            pltpu.SemaphoreType.DMA,
            pltpu.SemaphoreType.DMA,
            pltpu.SemaphoreType.DMA,
            pltpu.SemaphoreType.DMA,
        ],
    )
    return kfn(table, packed, zeros2)


def _a0_body(xr, wr, degr, out):
    dinv = lax.rsqrt(degr[0] + degr[1] + 1.0)
    out[...] = jnp.dot(xr[...], wr[...],
                       preferred_element_type=jnp.float32) * dinv


def _a_body(yr, sr, qr, gr, br, wr, degr, out):
    dinv = lax.rsqrt(degr[0] + degr[1] + 1.0)
    mu = sr[...] * (1.0 / NN)
    var = qr[...] * (1.0 / NN) - mu * mu
    a = gr[...] * (yr[...] - mu) * lax.rsqrt(var + 1e-5) + br[...]
    a = jnp.maximum(a, 0.0)
    out[...] = jnp.dot(a, wr[...], preferred_element_type=jnp.float32) * dinv


def _b_body(pr, hr, br, degr, y_out, s_out, q_out, *, stats):
    dinv = lax.rsqrt(degr[0] + degr[1] + 1.0)
    y = (pr[0] + pr[1] + hr[...]) * dinv + br[...]
    y_out[...] = y
    if stats:
        i = pl.program_id(0)

        @pl.when(i == 0)
        def _():
            s_out[...] = jnp.zeros_like(s_out)
            q_out[...] = jnp.zeros_like(q_out)

        s_out[...] += jnp.sum(y, axis=0, keepdims=True)
        q_out[...] += jnp.sum(y * y, axis=0, keepdims=True)


def _a0_call(xt, W, degp):
    return pl.pallas_call(
        _a0_body,
        grid=(GRID,),
        in_specs=[
            pl.BlockSpec((BLK, HH), lambda i: (i, 0)),
            pl.BlockSpec((HH, HH), lambda i: (0, 0)),
            pl.BlockSpec((NC, BLK, 1), lambda i: (0, i, 0)),
        ],
        out_specs=pl.BlockSpec((BLK, HH), lambda i: (i, 0)),
        out_shape=jax.ShapeDtypeStruct((NN, HH), jnp.float32),
    )(xt, W, degp)


def _a_call(y, s, q, g, be, W, degp):
    return pl.pallas_call(
        _a_body,
        grid=(GRID,),
        in_specs=[
            pl.BlockSpec((BLK, HH), lambda i: (i, 0)),
            pl.BlockSpec((1, HH), lambda i: (0, 0)),
            pl.BlockSpec((1, HH), lambda i: (0, 0)),
            pl.BlockSpec((1, HH), lambda i: (0, 0)),
            pl.BlockSpec((1, HH), lambda i: (0, 0)),
            pl.BlockSpec((HH, HH), lambda i: (0, 0)),
            pl.BlockSpec((NC, BLK, 1), lambda i: (0, i, 0)),
        ],
        out_specs=pl.BlockSpec((BLK, HH), lambda i: (i, 0)),
        out_shape=jax.ShapeDtypeStruct((NN, HH), jnp.float32),
    )(y, s, q, g, be, W, degp)


def _b_call(parts, h, b, degp, stats):
    outs = pl.pallas_call(
        functools.partial(_b_body, stats=stats),
        grid=(GRID,),
        in_specs=[
            pl.BlockSpec((NC, BLK, HH), lambda i: (0, i, 0)),
            pl.BlockSpec((BLK, HH), lambda i: (i, 0)),
            pl.BlockSpec((1, HH), lambda i: (0, 0)),
            pl.BlockSpec((NC, BLK, 1), lambda i: (0, i, 0)),
        ],
        out_specs=[
            pl.BlockSpec((BLK, HH), lambda i: (i, 0)),
            pl.BlockSpec((1, HH), lambda i: (0, 0)),
            pl.BlockSpec((1, HH), lambda i: (0, 0)),
        ],
        out_shape=[
            jax.ShapeDtypeStruct((NN, HH), jnp.float32),
            jax.ShapeDtypeStruct((1, HH), jnp.float32),
            jax.ShapeDtypeStruct((1, HH), jnp.float32),
        ],
    )(parts, h, b, degp)
    return outs


def kernel(x, edge_index, W0, b0, W1, b1, W2, b2, g0, be0, g1, be1):
    n = x.shape[1]
    xt = jnp.transpose(x, (1, 0, 2)).reshape(n, -1)
    padn = EPW - EE // NW
    src = edge_index[0].reshape(NW, EE // NW)
    dst = edge_index[1].reshape(NW, EE // NW)
    src = jnp.concatenate(
        [src, jnp.zeros((NW, padn), jnp.int32)], axis=1)
    dst = jnp.concatenate(
        [dst, jnp.full((NW, padn), NPAD - 1, jnp.int32)], axis=1)
    packed = ((src << 14) | dst).reshape(NW, NCHUNK, CH)
    dst = dst.reshape(NW, NCHUNK, CH)
    zeros1 = jnp.zeros((NPAD,), jnp.float32)
    zeros2 = jnp.zeros((NPAD, HH), jnp.float32)

    degp = _deg_call(dst, zeros1)
    degp = degp[:, :, None]

    b0r = b0.reshape(1, HH)
    b1r = b1.reshape(1, HH)
    b2r = b2.reshape(1, HH)

    h0 = _a0_call(xt, W0, degp)
    p0 = _edge_call(h0, packed, zeros2)
    y0, s0, q0 = _b_call(p0, h0, b0r, degp, stats=True)

    h1 = _a_call(y0, s0, q0, g0.reshape(1, HH), be0.reshape(1, HH), W1, degp)
    p1 = _edge_call(h1, packed, zeros2)
    y1, s1, q1 = _b_call(p1, h1, b1r, degp, stats=True)

    h2 = _a_call(y1, s1, q1, g1.reshape(1, HH), be1.reshape(1, HH), W2, degp)
    p2 = _edge_call(h2, packed, zeros2)
    y2, _, _ = _b_call(p2, h2, b2r, degp, stats=False)

    return jnp.transpose(y2, (1, 0))[:, :, None]


# speedup vs baseline: 1.0719x; 1.0719x over previous
import functools

import jax
import jax.numpy as jnp
from jax import lax
from jax.experimental import pallas as pl
from jax.experimental.pallas import tpu as pltpu
from jax.experimental.pallas import tpu_sc as plsc

NN = 10000
EE = 320000
HH = 128
NC = 2
NS = 16
NW = NC * NS
NPAD = 10240
EPW = 10240
CH = 80
NCHUNK = EPW // CH
RPS = NPAD // NS
BLK = 1000
GRID = NN // BLK

_mesh = plsc.VectorSubcoreMesh(core_axis_name="c", subcore_axis_name="s")


def _deg_body(dsts, zeros1, out, degsh, dstv, onesv):
    c = lax.axis_index("c")
    s = lax.axis_index("s")
    wid = c * NS + s
    for j in range(CH // 16):
        onesv[pl.ds(16 * j, 16)] = jnp.ones((16,), jnp.float32)
    pltpu.sync_copy(zeros1.at[pl.ds(s * RPS, RPS)], degsh.at[pl.ds(s * RPS, RPS)])
    pltpu.sync_copy(dsts.at[wid], dstv)
    plsc.subcore_barrier()

    def chunk(i, carry):
        pltpu.sync_copy(onesv, degsh.at[dstv.at[i]], add=True)
        return carry

    lax.fori_loop(0, NCHUNK, chunk, 0)
    plsc.subcore_barrier()
    pltpu.sync_copy(degsh.at[pl.ds(s * RPS, RPS)], out.at[c, pl.ds(s * RPS, RPS)])


def _deg_call(dst, zeros1):
    kfn = pl.kernel(
        _deg_body,
        out_type=jax.ShapeDtypeStruct((NC, NPAD), jnp.float32),
        mesh=_mesh,
        scratch_types=[
            pltpu.VMEM_SHARED((NPAD,), jnp.float32),
            pltpu.VMEM((NCHUNK, CH), jnp.int32),
            pltpu.VMEM((CH,), jnp.float32),
        ],
    )
    return kfn(dst, zeros1)


def _edge_body(table, packed, zeros2, out, accsh, pk, s0v, d0v, s1v, d1v,
               rows0, rows1, sem0, sem1):
    c = lax.axis_index("c")
    s = lax.axis_index("s")
    wid = c * NS + s
    pltpu.sync_copy(zeros2.at[pl.ds(s * RPS, RPS)], accsh.at[pl.ds(s * RPS, RPS)])
    pltpu.sync_copy(packed.at[wid], pk)
    plsc.subcore_barrier()

    def unpack(i, sv, dv):
        for j in range(CH // 16):
            p = pk[i, pl.ds(16 * j, 16)]
            sv[pl.ds(16 * j, 16)] = lax.shift_right_logical(p, 14)
            dv[pl.ds(16 * j, 16)] = lax.bitwise_and(p, 16383)

    unpack(0, s0v, d0v)
    pltpu.async_copy(table.at[s0v], rows0, sem0)
    unpack(1, s1v, d1v)
    pltpu.async_copy(table.at[s1v], rows1, sem1)

    def body(k, carry):
        i0 = 2 * k
        pltpu.make_async_copy(table.at[s0v], rows0, sem0).wait()
        pltpu.sync_copy(rows0, accsh.at[d0v], add=True)
        unpack(lax.rem(i0 + 2, NCHUNK), s0v, d0v)
        pltpu.async_copy(table.at[s0v], rows0, sem0)
        pltpu.make_async_copy(table.at[s1v], rows1, sem1).wait()
        pltpu.sync_copy(rows1, accsh.at[d1v], add=True)
        unpack(lax.rem(i0 + 3, NCHUNK), s1v, d1v)
        pltpu.async_copy(table.at[s1v], rows1, sem1)
        return carry

    lax.fori_loop(0, NCHUNK // 2, body, 0)
    pltpu.make_async_copy(table.at[s0v], rows0, sem0).wait()
    pltpu.make_async_copy(table.at[s1v], rows1, sem1).wait()
    plsc.subcore_barrier()
    pltpu.sync_copy(accsh.at[pl.ds(s * RPS, RPS)], out.at[c, pl.ds(s * RPS, RPS)])


def _edge_call(table, packed, zeros2):
    kfn = pl.kernel(
        _edge_body,
        out_type=jax.ShapeDtypeStruct((NC, NPAD, HH), jnp.float32),
        mesh=_mesh,
        scratch_types=[
            pltpu.VMEM_SHARED((NPAD, HH), jnp.float32),
            pltpu.VMEM((NCHUNK, CH), jnp.int32),
            pltpu.VMEM((CH,), jnp.int32),
            pltpu.VMEM((CH,), jnp.int32),
            pltpu.VMEM((CH,), jnp.int32),
            pltpu.VMEM((CH,), jnp.int32),
            pltpu.VMEM((CH, HH), jnp.float32),
            pltpu.VMEM((CH, HH), jnp.float32),
            pltpu.SemaphoreType.DMA,
            pltpu.SemaphoreType.DMA,
        ],
    )
    return kfn(table, packed, zeros2)


def _a0_body(xr, wr, degr, out):
    dinv = lax.rsqrt(degr[0] + degr[1] + 1.0)
    out[...] = jnp.dot(xr[...], wr[...],
                       preferred_element_type=jnp.float32) * dinv


def _a_body(yr, sr, qr, gr, br, wr, degr, out):
    dinv = lax.rsqrt(degr[0] + degr[1] + 1.0)
    mu = sr[...] * (1.0 / NN)
    var = qr[...] * (1.0 / NN) - mu * mu
    a = gr[...] * (yr[...] - mu) * lax.rsqrt(var + 1e-5) + br[...]
    a = jnp.maximum(a, 0.0)
    out[...] = jnp.dot(a, wr[...], preferred_element_type=jnp.float32) * dinv


def _b_body(pr, hr, br, degr, y_out, s_out, q_out, *, stats):
    dinv = lax.rsqrt(degr[0] + degr[1] + 1.0)
    y = (pr[0] + pr[1] + hr[...]) * dinv + br[...]
    y_out[...] = y
    if stats:
        i = pl.program_id(0)

        @pl.when(i == 0)
        def _():
            s_out[...] = jnp.zeros_like(s_out)
            q_out[...] = jnp.zeros_like(q_out)

        s_out[...] += jnp.sum(y, axis=0, keepdims=True)
        q_out[...] += jnp.sum(y * y, axis=0, keepdims=True)


def _a0_call(xt, W, degp):
    return pl.pallas_call(
        _a0_body,
        grid=(GRID,),
        in_specs=[
            pl.BlockSpec((BLK, HH), lambda i: (i, 0)),
            pl.BlockSpec((HH, HH), lambda i: (0, 0)),
            pl.BlockSpec((NC, BLK, 1), lambda i: (0, i, 0)),
        ],
        out_specs=pl.BlockSpec((BLK, HH), lambda i: (i, 0)),
        out_shape=jax.ShapeDtypeStruct((NN, HH), jnp.float32),
    )(xt, W, degp)


def _a_call(y, s, q, g, be, W, degp):
    return pl.pallas_call(
        _a_body,
        grid=(GRID,),
        in_specs=[
            pl.BlockSpec((BLK, HH), lambda i: (i, 0)),
            pl.BlockSpec((1, HH), lambda i: (0, 0)),
            pl.BlockSpec((1, HH), lambda i: (0, 0)),
            pl.BlockSpec((1, HH), lambda i: (0, 0)),
            pl.BlockSpec((1, HH), lambda i: (0, 0)),
            pl.BlockSpec((HH, HH), lambda i: (0, 0)),
            pl.BlockSpec((NC, BLK, 1), lambda i: (0, i, 0)),
        ],
        out_specs=pl.BlockSpec((BLK, HH), lambda i: (i, 0)),
        out_shape=jax.ShapeDtypeStruct((NN, HH), jnp.float32),
    )(y, s, q, g, be, W, degp)


def _b_call(parts, h, b, degp, stats):
    outs = pl.pallas_call(
        functools.partial(_b_body, stats=stats),
        grid=(GRID,),
        in_specs=[
            pl.BlockSpec((NC, BLK, HH), lambda i: (0, i, 0)),
            pl.BlockSpec((BLK, HH), lambda i: (i, 0)),
            pl.BlockSpec((1, HH), lambda i: (0, 0)),
            pl.BlockSpec((NC, BLK, 1), lambda i: (0, i, 0)),
        ],
        out_specs=[
            pl.BlockSpec((BLK, HH), lambda i: (i, 0)),
            pl.BlockSpec((1, HH), lambda i: (0, 0)),
            pl.BlockSpec((1, HH), lambda i: (0, 0)),
        ],
        out_shape=[
            jax.ShapeDtypeStruct((NN, HH), jnp.float32),
            jax.ShapeDtypeStruct((1, HH), jnp.float32),
            jax.ShapeDtypeStruct((1, HH), jnp.float32),
        ],
    )(parts, h, b, degp)
    return outs


def kernel(x, edge_index, W0, b0, W1, b1, W2, b2, g0, be0, g1, be1):
    n = x.shape[1]
    xt = jnp.transpose(x, (1, 0, 2)).reshape(n, -1)
    padn = EPW - EE // NW
    src = edge_index[0].reshape(NW, EE // NW)
    dst = edge_index[1].reshape(NW, EE // NW)
    src = jnp.concatenate(
        [src, jnp.zeros((NW, padn), jnp.int32)], axis=1)
    dst = jnp.concatenate(
        [dst, jnp.full((NW, padn), NPAD - 1, jnp.int32)], axis=1)
    packed = ((src << 14) | dst).reshape(NW, NCHUNK, CH)
    dst = dst.reshape(NW, NCHUNK, CH)
    zeros1 = jnp.zeros((NPAD,), jnp.float32)
    zeros2 = jnp.zeros((NPAD, HH), jnp.float32)

    degp = _deg_call(dst, zeros1)
    degp = degp[:, :, None]

    b0r = b0.reshape(1, HH)
    b1r = b1.reshape(1, HH)
    b2r = b2.reshape(1, HH)

    h0 = _a0_call(xt, W0, degp)
    p0 = _edge_call(h0, packed, zeros2)
    y0, s0, q0 = _b_call(p0, h0, b0r, degp, stats=True)

    h1 = _a_call(y0, s0, q0, g0.reshape(1, HH), be0.reshape(1, HH), W1, degp)
    p1 = _edge_call(h1, packed, zeros2)
    y1, s1, q1 = _b_call(p1, h1, b1r, degp, stats=True)

    h2 = _a_call(y1, s1, q1, g1.reshape(1, HH), be1.reshape(1, HH), W2, degp)
    p2 = _edge_call(h2, packed, zeros2)
    y2, _, _ = _b_call(p2, h2, b2r, degp, stats=False)

    return jnp.transpose(y2, (1, 0))[:, :, None]
